# baseline (speedup 1.0000x reference)
import functools

import jax
import jax.numpy as jnp
from jax.experimental import pallas as pl
from jax.experimental.pallas import tpu as pltpu


def _se_flat_kernel(x_ref, e_ref, w1a_ref, fold_ref, dmask_ref, b1_ref,
                    w2t_ref, b2_ref, idx_ref, o_ref, *, nb, g, cg):
    x = x_ref[...]
    R, L = x.shape
    C = g * cg
    cr = fold_ref.shape[1]

    pooled_r = jnp.dot(x, e_ref[...],
                       preferred_element_type=jnp.float32)

    p_all = jnp.dot(pooled_r, w1a_ref[...],
                    preferred_element_type=jnp.float32)
    p_all = p_all * dmask_ref[...]
    s_nb = jnp.sum(p_all.reshape(nb, g, g * cr), axis=1)
    h = jnp.dot(s_nb, fold_ref[...],
                preferred_element_type=jnp.float32) + b1_ref[...]
    h = jnp.maximum(h, 0.0)

    s = jnp.dot(h, w2t_ref[...], preferred_element_type=jnp.float32)
    gate = jax.nn.sigmoid(s + b2_ref[...])

    gate_rows = jnp.broadcast_to(
        gate[:, None, :], (nb, g, C)).reshape(R, C)
    row_iota = jax.lax.broadcasted_iota(jnp.int32, (R, 1), 0)
    qm = row_iota % (128 // cg)
    idx = qm * cg + idx_ref[...]
    n_half = C // 128
    if n_half <= 1:
        ge = jnp.take_along_axis(gate_rows, idx, axis=1)
    else:
        half_id = (row_iota % g) // (128 // cg)
        ge = jnp.take_along_axis(gate_rows[:, :128], idx, axis=1)
        for hh in range(1, n_half):
            ge_h = jnp.take_along_axis(
                gate_rows[:, hh * 128:(hh + 1) * 128], idx, axis=1)
            ge = jnp.where(half_id == hh, ge_h, ge)
    o_ref[...] = (x * ge).astype(o_ref.dtype)


def _se_native_kernel(x_ref, w1t_ref, b1_ref, w2t_ref, b2_ref, o_ref,
                      *, inv_hw):
    pooled = jnp.sum(x_ref[...], axis=-1, dtype=jnp.float32) * inv_hw
    h = jnp.dot(pooled, w1t_ref[...], preferred_element_type=jnp.float32)
    h = jnp.maximum(h + b1_ref[...], 0.0)
    s = jnp.dot(h, w2t_ref[...], preferred_element_type=jnp.float32)
    gate = jax.nn.sigmoid(s + b2_ref[...])
    o_ref[...] = (x_ref[...] * gate[:, :, None]).astype(o_ref.dtype)


def _pick_nb(N, per_batch_bytes, budget_bytes):
    cap = max(1, budget_bytes // per_batch_bytes)
    best = 1
    for nb in range(1, N + 1):
        if N % nb or nb > cap:
            continue
        if N // nb < 8 and N >= 8:
            continue
        best = nb
    return best


def kernel(x, w1, b1, w2, b2):
    N, C, H, W = x.shape
    Cr = w1.shape[0]
    HW = H * W

    w1t = jnp.transpose(w1.reshape(Cr, C))
    w2t = jnp.transpose(w2.reshape(C, Cr))
    b1r = b1.reshape(1, Cr)
    b2r = b2.reshape(1, C)

    g = 0
    for cand in (8, 16, 32, 64, 128):
        if C % cand:
            continue
        cgc = C // cand
        if ((cgc * HW) % 128 or cgc > 128 or 128 % cgc
                or C % 128 or cand % (128 // cgc)):
            continue
        g = cand
        break

    itemsize = jnp.dtype(x.dtype).itemsize
    if g > 0:
        cg = C // g
        L = cg * HW
        xv = x.reshape(N * g, L)

        per_batch = 4 * g * L * itemsize
        nb = _pick_nb(N, per_batch, 24 << 20)
        grid = (N // nb,)
        R = nb * g

        lane_c = jnp.arange(L, dtype=jnp.int32) // HW
        e_mat = (lane_c[:, None] == jnp.arange(cg, dtype=jnp.int32)[None, :]
                 ).astype(jnp.float32) * (1.0 / HW)
        idx = lane_c.reshape(1, L)

        w1a = jnp.transpose(w1t.reshape(g, cg, Cr), (1, 0, 2)
                            ).reshape(cg, g * Cr)
        qcol = jnp.arange(g * Cr, dtype=jnp.int32) // Cr
        dmask = (qcol[None, :] == (jnp.arange(R, dtype=jnp.int32) % g)[:, None]
                 ).astype(jnp.float32)
        fold = (jnp.arange(g * Cr, dtype=jnp.int32)[:, None] % Cr
                == jnp.arange(Cr, dtype=jnp.int32)[None, :]
                ).astype(jnp.float32)

        x_spec = pl.BlockSpec((R, L), lambda i: (i, 0))
        const_specs = [
            pl.BlockSpec((L, cg), lambda i: (0, 0)),
            pl.BlockSpec((cg, g * Cr), lambda i: (0, 0)),
            pl.BlockSpec((g * Cr, Cr), lambda i: (0, 0)),
            pl.BlockSpec((R, g * Cr), lambda i: (0, 0)),
            pl.BlockSpec((1, Cr), lambda i: (0, 0)),
            pl.BlockSpec((Cr, C), lambda i: (0, 0)),
            pl.BlockSpec((1, C), lambda i: (0, 0)),
            pl.BlockSpec((1, L), lambda i: (0, 0)),
        ]

        out_flat = pl.pallas_call(
            functools.partial(_se_flat_kernel, nb=nb, g=g, cg=cg),
            out_shape=jax.ShapeDtypeStruct((N * g, L), x.dtype),
            grid_spec=pl.GridSpec(
                grid=grid,
                in_specs=[x_spec] + const_specs,
                out_specs=x_spec,
            ),
            compiler_params=pltpu.CompilerParams(
                dimension_semantics=("parallel",),
                vmem_limit_bytes=min(nb * per_batch + (16 << 20), 48 << 20),
            ),
        )(xv, e_mat, w1a, fold, dmask, b1r, w2t, b2r, idx)
        return out_flat.reshape(N, C, H, W)

    x3 = x.reshape(N, C, HW)
    hw_pad = ((HW + 127) // 128) * 128
    per_batch = 4 * C * hw_pad * itemsize
    nb = _pick_nb(N, per_batch, 24 << 20)
    grid = (N // nb,)

    x_spec = pl.BlockSpec((nb, C, HW), lambda n: (n, 0, 0))
    w_specs = [
        pl.BlockSpec((C, Cr), lambda n: (0, 0)),
        pl.BlockSpec((1, Cr), lambda n: (0, 0)),
        pl.BlockSpec((Cr, C), lambda n: (0, 0)),
        pl.BlockSpec((1, C), lambda n: (0, 0)),
    ]
    out_flat = pl.pallas_call(
        functools.partial(_se_native_kernel, inv_hw=1.0 / HW),
        out_shape=jax.ShapeDtypeStruct((N, C, HW), x.dtype),
        grid_spec=pl.GridSpec(
            grid=grid,
            in_specs=[x_spec] + w_specs,
            out_specs=x_spec,
        ),
        compiler_params=pltpu.CompilerParams(
            dimension_semantics=("parallel",),
            vmem_limit_bytes=min(nb * per_batch + (8 << 20), 48 << 20),
        ),
    )(x3, w1t, b1r, w2t, b2r)
    return out_flat.reshape(N, C, H, W)


# speedup vs baseline: 22.3068x; 22.3068x over previous
import functools

import jax
import jax.numpy as jnp
from jax.experimental import pallas as pl
from jax.experimental.pallas import tpu as pltpu


def _se_hwnc_kernel(x_ref, w1t_ref, b1_ref, w2t_ref, b2_ref, o_ref, *, inv_hw):
    pooled = jnp.sum(x_ref[...], axis=0) * inv_hw

    h = jnp.dot(pooled, w1t_ref[...], preferred_element_type=jnp.float32)
    h = jnp.maximum(h + b1_ref[...], 0.0)
    s = jnp.dot(h, w2t_ref[...], preferred_element_type=jnp.float32)
    gate = jax.nn.sigmoid(s + b2_ref[...])

    o_ref[...] = (x_ref[...] * gate[None, :, :]).astype(o_ref.dtype)


def _se_native_kernel(x_ref, w1t_ref, b1_ref, w2t_ref, b2_ref, o_ref,
                      *, inv_hw):
    pooled = jnp.sum(x_ref[...], axis=-1, dtype=jnp.float32) * inv_hw
    h = jnp.dot(pooled, w1t_ref[...], preferred_element_type=jnp.float32)
    h = jnp.maximum(h + b1_ref[...], 0.0)
    s = jnp.dot(h, w2t_ref[...], preferred_element_type=jnp.float32)
    gate = jax.nn.sigmoid(s + b2_ref[...])
    o_ref[...] = (x_ref[...] * gate[:, :, None]).astype(o_ref.dtype)


def _pick_nb(N, per_batch_bytes, budget_bytes, min_steps):
    cap = max(1, budget_bytes // per_batch_bytes)
    best = 1
    for nb in range(1, N + 1):
        if N % nb or nb > cap:
            continue
        if N // nb < min_steps and N >= min_steps:
            continue
        best = nb
    return best


def kernel(x, w1, b1, w2, b2):
    N, C, H, W = x.shape
    Cr = w1.shape[0]
    HW = H * W
    itemsize = jnp.dtype(x.dtype).itemsize

    w1t = jnp.transpose(w1.reshape(Cr, C))
    w2t = jnp.transpose(w2.reshape(C, Cr))
    b1r = b1.reshape(1, Cr)
    b2r = b2.reshape(1, C)
    w_specs = [
        pl.BlockSpec((C, Cr), lambda n: (0, 0)),
        pl.BlockSpec((1, Cr), lambda n: (0, 0)),
        pl.BlockSpec((Cr, C), lambda n: (0, 0)),
        pl.BlockSpec((1, C), lambda n: (0, 0)),
    ]

    if N % 8 == 0 and C % 128 == 0:
        y = jnp.transpose(x.reshape(N, C, HW), (2, 0, 1))

        per_batch = 4 * HW * C * itemsize
        nb = _pick_nb(N, per_batch, 28 << 20, 8)
        grid = (N // nb,)

        y_spec = pl.BlockSpec((HW, nb, C), lambda n: (0, n, 0))
        out_t = pl.pallas_call(
            functools.partial(_se_hwnc_kernel, inv_hw=1.0 / HW),
            out_shape=jax.ShapeDtypeStruct((HW, N, C), x.dtype),
            grid_spec=pl.GridSpec(
                grid=grid,
                in_specs=[y_spec] + w_specs,
                out_specs=y_spec,
            ),
            compiler_params=pltpu.CompilerParams(
                dimension_semantics=("parallel",),
                vmem_limit_bytes=min(nb * per_batch + (8 << 20), 56 << 20),
            ),
        )(y, w1t, b1r, w2t, b2r)
        return out_t.transpose(1, 2, 0).reshape(N, C, H, W)

    x3 = x.reshape(N, C, HW)
    hw_pad = ((HW + 127) // 128) * 128
    per_batch = 4 * C * hw_pad * itemsize
    nb = _pick_nb(N, per_batch, 24 << 20, 4)
    grid = (N // nb,)

    x_spec = pl.BlockSpec((nb, C, HW), lambda n: (n, 0, 0))
    out_flat = pl.pallas_call(
        functools.partial(_se_native_kernel, inv_hw=1.0 / HW),
        out_shape=jax.ShapeDtypeStruct((N, C, HW), x.dtype),
        grid_spec=pl.GridSpec(
            grid=grid,
            in_specs=[x_spec] + w_specs,
            out_specs=x_spec,
        ),
        compiler_params=pltpu.CompilerParams(
            dimension_semantics=("parallel",),
            vmem_limit_bytes=min(nb * per_batch + (8 << 20), 48 << 20),
        ),
    )(x3, w1t, b1r, w2t, b2r)
    return out_flat.reshape(N, C, H, W)


# speedup vs baseline: 24.6028x; 1.1029x over previous
import functools

import jax
import jax.numpy as jnp
from jax.experimental import pallas as pl
from jax.experimental.pallas import tpu as pltpu


def _se_hwnc_kernel(x_ref, w1t_ref, b1_ref, w2t_ref, b2_ref, o_ref, *, inv_hw):
    pooled = jnp.sum(x_ref[...], axis=0) * inv_hw

    h = jnp.dot(pooled, w1t_ref[...], preferred_element_type=jnp.float32)
    h = jnp.maximum(h + b1_ref[...], 0.0)
    s = jnp.dot(h, w2t_ref[...], preferred_element_type=jnp.float32)
    gate = jax.nn.sigmoid(s + b2_ref[...])

    o_ref[...] = (x_ref[...] * gate[None, :, :]).astype(o_ref.dtype)


def _se_native_kernel(x_ref, w1t_ref, b1_ref, w2t_ref, b2_ref, o_ref,
                      *, inv_hw):
    pooled = jnp.sum(x_ref[...], axis=-1, dtype=jnp.float32) * inv_hw
    h = jnp.dot(pooled, w1t_ref[...], preferred_element_type=jnp.float32)
    h = jnp.maximum(h + b1_ref[...], 0.0)
    s = jnp.dot(h, w2t_ref[...], preferred_element_type=jnp.float32)
    gate = jax.nn.sigmoid(s + b2_ref[...])
    o_ref[...] = (x_ref[...] * gate[:, :, None]).astype(o_ref.dtype)


def _pick_nb(N, per_batch_bytes, budget_bytes, min_steps):
    cap = max(1, budget_bytes // per_batch_bytes)
    best = 1
    for nb in range(1, N + 1):
        if N % nb or nb > cap:
            continue
        if N // nb < min_steps and N >= min_steps:
            continue
        best = nb
    return best


def kernel(x, w1, b1, w2, b2):
    N, C, H, W = x.shape
    Cr = w1.shape[0]
    HW = H * W
    itemsize = jnp.dtype(x.dtype).itemsize

    w1t = jnp.transpose(w1.reshape(Cr, C))
    w2t = jnp.transpose(w2.reshape(C, Cr))
    b1r = b1.reshape(1, Cr)
    b2r = b2.reshape(1, C)
    w_specs = [
        pl.BlockSpec((C, Cr), lambda n: (0, 0)),
        pl.BlockSpec((1, Cr), lambda n: (0, 0)),
        pl.BlockSpec((Cr, C), lambda n: (0, 0)),
        pl.BlockSpec((1, C), lambda n: (0, 0)),
    ]

    if N % 8 == 0 and C % 128 == 0:
        y = jnp.transpose(x.reshape(N, C, HW), (2, 0, 1))

        per_batch = 4 * HW * C * itemsize
        nb = _pick_nb(N, per_batch, 28 << 20, 4)
        grid = (N // nb,)

        y_spec = pl.BlockSpec((HW, nb, C), lambda n: (0, n, 0))
        out_t = pl.pallas_call(
            functools.partial(_se_hwnc_kernel, inv_hw=1.0 / HW),
            out_shape=jax.ShapeDtypeStruct((HW, N, C), x.dtype),
            grid_spec=pl.GridSpec(
                grid=grid,
                in_specs=[y_spec] + w_specs,
                out_specs=y_spec,
            ),
            compiler_params=pltpu.CompilerParams(
                dimension_semantics=("parallel",),
                vmem_limit_bytes=min(nb * per_batch + (8 << 20), 56 << 20),
            ),
        )(y, w1t, b1r, w2t, b2r)
        return out_t.transpose(1, 2, 0).reshape(N, C, H, W)

    x3 = x.reshape(N, C, HW)
    hw_pad = ((HW + 127) // 128) * 128
    per_batch = 4 * C * hw_pad * itemsize
    nb = _pick_nb(N, per_batch, 24 << 20, 4)
    grid = (N // nb,)

    x_spec = pl.BlockSpec((nb, C, HW), lambda n: (n, 0, 0))
    out_flat = pl.pallas_call(
        functools.partial(_se_native_kernel, inv_hw=1.0 / HW),
        out_shape=jax.ShapeDtypeStruct((N, C, HW), x.dtype),
        grid_spec=pl.GridSpec(
            grid=grid,
            in_specs=[x_spec] + w_specs,
            out_specs=x_spec,
        ),
        compiler_params=pltpu.CompilerParams(
            dimension_semantics=("parallel",),
            vmem_limit_bytes=min(nb * per_batch + (8 << 20), 48 << 20),
        ),
    )(x3, w1t, b1r, w2t, b2r)
    return out_flat.reshape(N, C, H, W)


# speedup vs baseline: 27.1263x; 1.1026x over previous
import functools

import jax
import jax.numpy as jnp
from jax.experimental import pallas as pl
from jax.experimental.pallas import tpu as pltpu


def _se_hwnc_kernel(x_ref, w1t_ref, b1_ref, w2t_ref, b2_ref, o_ref, *, inv_hw):
    pooled = jnp.sum(x_ref[...], axis=0) * inv_hw

    h = jnp.dot(pooled, w1t_ref[...], preferred_element_type=jnp.float32)
    h = jnp.maximum(h + b1_ref[...], 0.0)
    s = jnp.dot(h, w2t_ref[...], preferred_element_type=jnp.float32)
    gate = jax.nn.sigmoid(s + b2_ref[...])

    o_ref[...] = (x_ref[...] * gate[None, :, :]).astype(o_ref.dtype)


def _se_native_kernel(x_ref, w1t_ref, b1_ref, w2t_ref, b2_ref, o_ref,
                      *, inv_hw):
    pooled = jnp.sum(x_ref[...], axis=-1, dtype=jnp.float32) * inv_hw
    h = jnp.dot(pooled, w1t_ref[...], preferred_element_type=jnp.float32)
    h = jnp.maximum(h + b1_ref[...], 0.0)
    s = jnp.dot(h, w2t_ref[...], preferred_element_type=jnp.float32)
    gate = jax.nn.sigmoid(s + b2_ref[...])
    o_ref[...] = (x_ref[...] * gate[:, :, None]).astype(o_ref.dtype)


def _pick_nb(N, per_batch_bytes, budget_bytes, min_steps):
    cap = max(1, budget_bytes // per_batch_bytes)
    best = 1
    for nb in range(1, N + 1):
        if N % nb or nb > cap:
            continue
        if N // nb < min_steps and N >= min_steps:
            continue
        best = nb
    return best


def kernel(x, w1, b1, w2, b2):
    N, C, H, W = x.shape
    Cr = w1.shape[0]
    HW = H * W
    itemsize = jnp.dtype(x.dtype).itemsize

    w1t = jnp.transpose(w1.reshape(Cr, C))
    w2t = jnp.transpose(w2.reshape(C, Cr))
    b1r = b1.reshape(1, Cr)
    b2r = b2.reshape(1, C)
    w_specs = [
        pl.BlockSpec((C, Cr), lambda n: (0, 0)),
        pl.BlockSpec((1, Cr), lambda n: (0, 0)),
        pl.BlockSpec((Cr, C), lambda n: (0, 0)),
        pl.BlockSpec((1, C), lambda n: (0, 0)),
    ]

    if N % 8 == 0 and C % 128 == 0:
        y = jnp.transpose(x.reshape(N, C, HW), (2, 0, 1))

        per_batch = 4 * HW * C * itemsize
        nb = _pick_nb(N, per_batch, 54 << 20, 2)
        grid = (N // nb,)

        y_spec = pl.BlockSpec((HW, nb, C), lambda n: (0, n, 0))
        out_t = pl.pallas_call(
            functools.partial(_se_hwnc_kernel, inv_hw=1.0 / HW),
            out_shape=jax.ShapeDtypeStruct((HW, N, C), x.dtype),
            grid_spec=pl.GridSpec(
                grid=grid,
                in_specs=[y_spec] + w_specs,
                out_specs=y_spec,
            ),
            compiler_params=pltpu.CompilerParams(
                dimension_semantics=("parallel",),
                vmem_limit_bytes=min(nb * per_batch + (8 << 20), 60 << 20),
            ),
        )(y, w1t, b1r, w2t, b2r)
        return out_t.transpose(1, 2, 0).reshape(N, C, H, W)

    x3 = x.reshape(N, C, HW)
    hw_pad = ((HW + 127) // 128) * 128
    per_batch = 4 * C * hw_pad * itemsize
    nb = _pick_nb(N, per_batch, 24 << 20, 4)
    grid = (N // nb,)

    x_spec = pl.BlockSpec((nb, C, HW), lambda n: (n, 0, 0))
    out_flat = pl.pallas_call(
        functools.partial(_se_native_kernel, inv_hw=1.0 / HW),
        out_shape=jax.ShapeDtypeStruct((N, C, HW), x.dtype),
        grid_spec=pl.GridSpec(
            grid=grid,
            in_specs=[x_spec] + w_specs,
            out_specs=x_spec,
        ),
        compiler_params=pltpu.CompilerParams(
            dimension_semantics=("parallel",),
            vmem_limit_bytes=min(nb * per_batch + (8 << 20), 48 << 20),
        ),
    )(x3, w1t, b1r, w2t, b2r)
    return out_flat.reshape(N, C, H, W)
